# speedup vs baseline: 1.1325x; 1.1325x over previous
import functools

import jax
import jax.numpy as jnp
from jax import lax
from jax.experimental import pallas as pl
from jax.experimental.pallas import tpu as pltpu
from jax.experimental.pallas import tpu_sc as plsc

_L = 16


def _sigmoid(x):
    return 1.0 / (1.0 + jnp.exp(-x))


@functools.lru_cache(maxsize=None)
def _make_sc_kernel(B: int, NC: int, NS: int):
    NW = NC * NS
    assert B % (NW * _L) == 0, (B, NW)
    bpw = B // NW
    mesh = plsc.VectorSubcoreMesh(core_axis_name="c", subcore_axis_name="s")

    @functools.partial(
        pl.kernel,
        mesh=mesh,
        out_type=jax.ShapeDtypeStruct((B,), jnp.float32),
        scratch_types=[
            pltpu.VMEM((bpw,), jnp.int32),
            pltpu.VMEM((bpw,), jnp.int32),
            pltpu.VMEM((bpw,), jnp.float32),
            pltpu.VMEM((bpw,), jnp.float32),
            pltpu.VMEM((bpw,), jnp.float32),
            pltpu.VMEM((bpw,), jnp.float32),
            pltpu.SemaphoreType.DMA,
            pltpu.SemaphoreType.DMA,
            pltpu.SemaphoreType.DMA,
        ],
    )
    def sc_kernel(stu_id, exer_id, s_tab, k_tab, d_tab, out,
                  sidx, eidx, srow, krow, drow, oval, sem0, sem1, sem2):
        wid = lax.axis_index("s") * NC + lax.axis_index("c")
        base = wid * bpw
        pltpu.sync_copy(stu_id.at[pl.ds(base, bpw)], sidx)
        pltpu.sync_copy(exer_id.at[pl.ds(base, bpw)], eidx)
        c0 = pltpu.async_copy(s_tab.at[sidx], srow, sem0)
        c1 = pltpu.async_copy(k_tab.at[eidx], krow, sem1)
        c2 = pltpu.async_copy(d_tab.at[eidx], drow, sem2)
        c0.wait()
        c1.wait()
        c2.wait()
        for j in range(bpw // _L):
            sl = pl.ds(j * _L, _L)
            s = _sigmoid(srow[sl])
            k = _sigmoid(krow[sl])
            e = _sigmoid(drow[sl]) * 10.0
            oval[sl] = _sigmoid(e * (s - k))
        pltpu.sync_copy(oval, out.at[pl.ds(base, bpw)])

    return sc_kernel


def kernel(stu_id, exer_id, student_emb, k_difficulty, e_discrimination):
    B = stu_id.shape[0]
    info = plsc.get_sparse_core_info()
    sc = _make_sc_kernel(B, info.num_cores, info.num_subcores)
    out = sc(
        stu_id.astype(jnp.int32),
        exer_id.astype(jnp.int32),
        student_emb.reshape(-1),
        k_difficulty.reshape(-1),
        e_discrimination.reshape(-1),
    )
    return out.reshape(B, 1)
